# baseline (speedup 1.0000x reference)
import functools

import jax
import jax.numpy as jnp
from jax import lax
from jax.experimental import pallas as pl
from jax.experimental.pallas import tpu as pltpu
from jax.experimental.pallas import tpu_sc as plsc

N = 10000
E = 320000
F = 128
H = 128
R = 4
L = 10

NC = 2
NS = 16
NW = NC * NS
CK = 32
NCH = 316
EP = NW * NCH * CK
NA = NCH * CK
NQ = 20
ZROW = NW * NA + NW
NRUNS = NW * NA + NW + 16


NV = H // 16


def _sc_agg_body(xr_hbm, gidx_hbm, new_hbm, flidx_hbm, slota_hbm, slotb_hbm,
                 out_hbm, runs_hbm,
                 gidx_v, new_v, flidx_v, sla_v, slb_v, rows_v, rowsb_v, fb_v,
                 state_v, sem):
    c = lax.axis_index("c")
    s = lax.axis_index("s")
    w = c * NS + s

    pltpu.sync_copy(gidx_hbm.at[w], gidx_v)
    pltpu.sync_copy(new_hbm.at[w], new_v)
    pltpu.sync_copy(flidx_hbm.at[w], flidx_v)
    pltpu.sync_copy(slota_hbm.at[c, s], sla_v)
    pltpu.sync_copy(slotb_hbm.at[c, s], slb_v)

    zeros16 = jnp.zeros((16,), jnp.float32)

    def zero_body(i, _):
        j = i // NV
        k = i % NV
        fb_v[j, pl.ds(k * 16, 16)] = zeros16
        state_v[j, pl.ds(k * 16, 16)] = zeros16
        return 0

    lax.fori_loop(0, CK * NV, zero_body, 0)

    @pl.when(s == 0)
    def _():
        z0 = pl.multiple_of(ZROW + 8 * c, 8)
        pltpu.sync_copy(fb_v.at[pl.ds(0, 8)], runs_hbm.at[pl.ds(z0, 8)])

    def chunk_body(j, _):
        pltpu.async_copy(xr_hbm.at[gidx_v.at[j]], rows_v, sem).wait()

        for g in range(CK // 16):
            nvec = new_v[pl.ds(j * CK + g * 16, 16)]
            for t in range(16):
                e = g * 16 + t
                ep = (e - 1) % CK
                keep = 1.0 - jnp.full((16,), nvec[t], jnp.float32)
                for k in range(NV):
                    r = rows_v[e, pl.ds(k * 16, 16)]
                    a = state_v[ep, pl.ds(k * 16, 16)]
                    state_v[e, pl.ds(k * 16, 16)] = a * keep + r

        pltpu.sync_copy(state_v, runs_hbm.at[flidx_v.at[j]])
        return 0

    lax.fori_loop(0, NCH, chunk_body, 0)

    plsc.subcore_barrier()

    def q_body(q, _):
        r0 = pl.multiple_of(s * NQ * CK + q * CK, 8)
        pltpu.async_copy(runs_hbm.at[sla_v.at[q]], rows_v, sem).wait()
        pltpu.async_copy(runs_hbm.at[slb_v.at[q]], rowsb_v, sem).wait()
        for e in range(CK):
            for k in range(NV):
                state_v[e, pl.ds(k * 16, 16)] = (
                    rows_v[e, pl.ds(k * 16, 16)]
                    + rowsb_v[e, pl.ds(k * 16, 16)])

        @pl.when(r0 + CK <= N)
        def _():
            pltpu.sync_copy(state_v, out_hbm.at[c, pl.ds(r0, CK)])

        @pl.when(jnp.logical_and(r0 < N, r0 + CK > N))
        def _():
            tail = N % CK
            pltpu.sync_copy(state_v.at[pl.ds(0, tail)],
                            out_hbm.at[c, pl.ds(r0, tail)])
        return 0

    lax.fori_loop(0, NQ, q_body, 0)


_sc_agg = functools.partial(
    pl.kernel,
    mesh=plsc.VectorSubcoreMesh(core_axis_name="c", subcore_axis_name="s"),
    out_type=[
        jax.ShapeDtypeStruct((NC, N, H), jnp.float32),
        jax.ShapeDtypeStruct((NRUNS, H), jnp.float32),
    ],
    scratch_types=[
        pltpu.VMEM((NCH, CK), jnp.int32),
        pltpu.VMEM((NCH * CK,), jnp.float32),
        pltpu.VMEM((NCH, CK), jnp.int32),
        pltpu.VMEM((NQ, CK), jnp.int32),
        pltpu.VMEM((NQ, CK), jnp.int32),
        pltpu.VMEM((CK, H), jnp.float32),
        pltpu.VMEM((CK, H), jnp.float32),
        pltpu.VMEM((CK, H), jnp.float32),
        pltpu.VMEM((CK, H), jnp.float32),
        pltpu.SemaphoreType.DMA,
    ],
)(_sc_agg_body)


BN = 1000


def _mm(a, b):
    return jnp.dot(a, b, preferred_element_type=jnp.float32)


def _initpre_body(nf_ref, win_ref, bin_ref, wrel_ref, x_ref, xr_ref):
    x = jnp.tanh(_mm(nf_ref[...], win_ref[...]) + bin_ref[...])
    x_ref[...] = x
    for r in range(R):
        xr_ref[r] = _mm(x, wrel_ref[r])


def _initpre(nodes_fea, W_in, b_in, W_rel0):
    return pl.pallas_call(
        _initpre_body,
        grid=(N // BN,),
        in_specs=[
            pl.BlockSpec((BN, F), lambda i: (i, 0)),
            pl.BlockSpec((F, H), lambda i: (0, 0)),
            pl.BlockSpec((1, H), lambda i: (0, 0)),
            pl.BlockSpec((R, H, H), lambda i: (0, 0, 0)),
        ],
        out_specs=[
            pl.BlockSpec((BN, H), lambda i: (i, 0)),
            pl.BlockSpec((R, BN, H), lambda i: (0, i, 0)),
        ],
        out_shape=[
            jax.ShapeDtypeStruct((N, H), jnp.float32),
            jax.ShapeDtypeStruct((R, N, H), jnp.float32),
        ],
    )(nodes_fea, W_in, b_in, W_rel0)


def _update_core(acc_ref, x_ref, wloop_ref, brel_ref, w1_ref, b1_ref, w2_ref, b2_ref):
    x = x_ref[...]
    msg = (acc_ref[0] + acc_ref[1]) + _mm(x, wloop_ref[...]) + brel_ref[...]
    mid = jnp.tanh(_mm(jnp.concatenate([x, msg], axis=1), w1_ref[...]) + b1_ref[...])
    xn = jnp.tanh(_mm(jnp.concatenate([x, mid], axis=1), w2_ref[...]) + b2_ref[...])
    return xn


def _postpre_body(acc_ref, x_ref, wloop_ref, brel_ref, w1_ref, b1_ref, w2_ref,
                  b2_ref, wrel_ref, xn_ref, xr_ref):
    xn = _update_core(acc_ref, x_ref, wloop_ref, brel_ref, w1_ref, b1_ref,
                      w2_ref, b2_ref)
    xn_ref[...] = xn
    for r in range(R):
        xr_ref[r] = _mm(xn, wrel_ref[r])


def _post_body(acc_ref, x_ref, wloop_ref, brel_ref, w1_ref, b1_ref, w2_ref,
               b2_ref, xn_ref):
    xn_ref[...] = _update_core(acc_ref, x_ref, wloop_ref, brel_ref, w1_ref,
                               b1_ref, w2_ref, b2_ref)


_UPDATE_IN_SPECS = [
    pl.BlockSpec((NC, BN, H), lambda i: (0, i, 0)),
    pl.BlockSpec((BN, H), lambda i: (i, 0)),
    pl.BlockSpec((H, H), lambda i: (0, 0)),
    pl.BlockSpec((1, H), lambda i: (0, 0)),
    pl.BlockSpec((2 * H, 2 * H), lambda i: (0, 0)),
    pl.BlockSpec((1, 2 * H), lambda i: (0, 0)),
    pl.BlockSpec((3 * H, H), lambda i: (0, 0)),
    pl.BlockSpec((1, H), lambda i: (0, 0)),
]


def _postpre(acc2, x, W_loop_l, b_rel_l, W1_l, b1_l, W2_l, b2_l, W_rel_n):
    return pl.pallas_call(
        _postpre_body,
        grid=(N // BN,),
        in_specs=_UPDATE_IN_SPECS + [pl.BlockSpec((R, H, H), lambda i: (0, 0, 0))],
        out_specs=[
            pl.BlockSpec((BN, H), lambda i: (i, 0)),
            pl.BlockSpec((R, BN, H), lambda i: (0, i, 0)),
        ],
        out_shape=[
            jax.ShapeDtypeStruct((N, H), jnp.float32),
            jax.ShapeDtypeStruct((R, N, H), jnp.float32),
        ],
    )(acc2, x, W_loop_l, b_rel_l, W1_l, b1_l, W2_l, b2_l, W_rel_n)


def _post(acc2, x, W_loop_l, b_rel_l, W1_l, b1_l, W2_l, b2_l):
    return pl.pallas_call(
        _post_body,
        grid=(N // BN,),
        in_specs=_UPDATE_IN_SPECS,
        out_specs=pl.BlockSpec((BN, H), lambda i: (i, 0)),
        out_shape=jax.ShapeDtypeStruct((N, H), jnp.float32),
    )(acc2, x, W_loop_l, b_rel_l, W1_l, b1_l, W2_l, b2_l)


def kernel(nodes_fea, edges, edges_type, W_in, b_in, W_rel, b_rel, W_loop, W1, b1, W2, b2):
    src = edges[0]
    dst = edges[1]
    order = jnp.argsort(dst, stable=True)
    pad = EP - E
    gidx3 = jnp.concatenate(
        [(edges_type * N + src)[order], jnp.zeros((pad,), jnp.int32)]).reshape(NW, NCH, CK)
    dsts = jnp.concatenate(
        [dst[order], jnp.full((pad,), N, jnp.int32)]).reshape(NW, NA)
    isnew = jnp.concatenate(
        [jnp.ones((NW, 1), jnp.bool_), dsts[:, 1:] != dsts[:, :-1]], axis=1)
    runrank = jnp.cumsum(isnew.astype(jnp.int32), axis=1) - 1
    tile_id = jax.lax.broadcasted_iota(jnp.int32, (NW, NA), 0)
    new3 = isnew.astype(jnp.float32).reshape(NW, NA)
    isend = jnp.concatenate(
        [isnew[:, 1:], jnp.ones((NW, 1), jnp.bool_)], axis=1)
    gslot = tile_id * NA + runrank
    trash = NW * NA + tile_id
    flidx3 = jnp.where(isend, gslot, trash).reshape(NW, NCH, CK)
    fvals = (jnp.full((NW * NA,), N, jnp.int32)
             .at[gslot.reshape(-1)]
             .set(dsts.reshape(-1), indices_are_sorted=True))
    slot_ids = jnp.arange(NW * NA, dtype=jnp.int32)
    sid = (slot_ids // (NS * NA)) * (N + 1) + fvals
    sa = jax.ops.segment_min(slot_ids, sid, num_segments=2 * (N + 1))
    sb = jax.ops.segment_max(slot_ids, sid, num_segments=2 * (N + 1))
    NPAD = NS * NQ * CK
    slota = []
    slotb = []
    for cc in range(NC):
        zslot = ZROW + 8 * cc
        a = sa[cc * (N + 1): cc * (N + 1) + N]
        b = sb[cc * (N + 1): cc * (N + 1) + N]
        a = jnp.where(a >= NW * NA, zslot, a)
        b = jnp.where(jnp.logical_or(b < 0, b == a), zslot, b)
        pad = jnp.full((NPAD - N,), zslot, jnp.int32)
        slota.append(jnp.concatenate([a, pad]))
        slotb.append(jnp.concatenate([b, pad]))
    slota4 = jnp.stack(slota).reshape(NC, NS, NQ, CK)
    slotb4 = jnp.stack(slotb).reshape(NC, NS, NQ, CK)

    x, xr = _initpre(nodes_fea, W_in, b_in.reshape(1, H), W_rel[0])
    for l in range(L):
        acc2, _ = _sc_agg(xr.reshape(R * N, H), gidx3, new3, flidx3, slota4, slotb4)
        args = (acc2, x, W_loop[l], b_rel[l].reshape(1, H), W1[l],
                b1[l].reshape(1, 2 * H), W2[l], b2[l].reshape(1, H))
        if l < L - 1:
            x, xr = _postpre(*args, W_rel[l + 1])
        else:
            x = _post(*args)
    return x
